# Initial kernel scaffold; baseline (speedup 1.0000x reference)
import jax
import jax.numpy as jnp
from jax.experimental import pallas as pl


def kernel(cls_scores, bbox_deltas):
    raise NotImplementedError("write your pallas kernel here")


# speedup vs baseline: 45.6111x; 45.6111x over previous
import functools

import numpy as np
import jax
import jax.numpy as jnp
from jax.experimental import pallas as pl
from jax.experimental.pallas import tpu as pltpu

_IMAGE_SIZE = 1920
_NMS_PRE = 3000
_NMS_POST = 300
_THRESH = 0.6
_RATIOS = (0.5, 1.0, 2.0)
_SCALES = (8, 16, 32)
_H = 120
_W = 120
_K = 9
_N = _H * _W * _K
_ROWS = 8
_COLS = 16512
_PAD = _ROWS * _COLS
_BIG = np.int32(2**31 - 1)


def _gen_anchors():
    feat_stride, size = 16, _W
    shifts = np.arange(size, dtype=np.float32) * feat_stride + feat_stride / 2.0
    cy = shifts[:, None, None]
    cx = shifts[None, :, None]
    ws, hs = [], []
    for r in _RATIOS:
        for s in _SCALES:
            base = float(s) * float(feat_stride)
            ws.append(base * np.sqrt(1.0 / r))
            hs.append(base * np.sqrt(r))
    ws = np.asarray(ws, np.float32)[None, None, :]
    hs = np.asarray(hs, np.float32)[None, None, :]
    x1 = np.broadcast_to(cx - ws / 2.0, (size, size, _K))
    y1 = np.broadcast_to(cy - hs / 2.0, (size, size, _K))
    x2 = np.broadcast_to(cx + ws / 2.0, (size, size, _K))
    y2 = np.broadcast_to(cy + hs / 2.0, (size, size, _K))
    return np.stack([x1, y1, x2, y2], axis=-1).astype(np.float32)


def _scrambled_anchor_planes():
    a = _gen_anchors().reshape(_K, 4, _H, _W)
    planes = []
    for c in range(4):
        p = a[:, c, :, :].transpose(1, 2, 0).reshape(_N)
        planes.append(np.pad(p, (0, _PAD - _N)).reshape(_ROWS, _COLS))
    return planes


_ANCHOR_PLANES = _scrambled_anchor_planes()


def _nms_kernel(s_ref, d1_ref, d2_ref, d3_ref, d4_ref,
                a1_ref, a2_ref, a3_ref, a4_ref, out_ref,
                sw_ref, x1_ref, y1_ref, x2_ref, y2_ref, ar_ref):
    f32 = jnp.float32
    neg_inf = f32(-jnp.inf)

    x1 = jnp.clip(a1_ref[...] + d1_ref[...], 0.0, float(_IMAGE_SIZE))
    y1 = jnp.clip(a2_ref[...] + d2_ref[...], 0.0, float(_IMAGE_SIZE))
    x2 = jnp.clip(a3_ref[...] + d3_ref[...], 0.0, float(_IMAGE_SIZE))
    y2 = jnp.clip(a4_ref[...] + d4_ref[...], 0.0, float(_IMAGE_SIZE))
    x1_ref[...] = x1
    y1_ref[...] = y1
    x2_ref[...] = x2
    y2_ref[...] = y2
    ar_ref[...] = jnp.maximum(x2 - x1, 0.0) * jnp.maximum(y2 - y1, 0.0)

    idx = (jax.lax.broadcasted_iota(jnp.int32, (_ROWS, _COLS), 0) * _COLS
           + jax.lax.broadcasted_iota(jnp.int32, (_ROWS, _COLS), 1))

    s = s_ref[...]
    bits = jax.lax.bitcast_convert_type(s, jnp.int32)
    mapped = jnp.where(bits < 0, ~bits, bits ^ jnp.int32(-2**31))
    u = jax.lax.bitcast_convert_type(mapped, jnp.uint32)

    def _bit_step(i, t):
        cand = t | (jnp.uint32(1) << (jnp.uint32(31) - i.astype(jnp.uint32)))
        cnt = jnp.sum((u >= cand).astype(jnp.int32))
        return jnp.where(cnt >= _NMS_PRE, cand, t)

    t = jax.lax.fori_loop(0, 32, _bit_step, jnp.uint32(0))

    c1 = jnp.sum((u > t).astype(jnp.int32))
    m = _NMS_PRE - c1

    tie = u == t

    def _tie_step(_, lohi):
        lo, hi = lohi
        mid = (lo + hi) // 2
        cnt = jnp.sum((tie & (idx <= mid)).astype(jnp.int32))
        p = cnt >= m
        return jnp.where(p, lo, mid + 1), jnp.where(p, mid, hi)

    lo, _ = jax.lax.fori_loop(0, 18, _tie_step, (jnp.int32(0), jnp.int32(_N - 1)))
    eligible = (u > t) | (tie & (idx <= lo))
    sw_ref[...] = jnp.where(eligible, s, neg_inf)

    out_ref[...] = jnp.zeros((_NMS_POST, 8), f32)
    orow = jax.lax.broadcasted_iota(jnp.int32, (_NMS_POST, 8), 0)
    ocol = jax.lax.broadcasted_iota(jnp.int32, (_NMS_POST, 8), 1)

    def _greedy(i, _):
        sw = sw_ref[...]
        ms = jnp.max(sw)

        @pl.when(ms > neg_inf)
        def _():
            sel0 = sw == ms
            cidx = jnp.min(jnp.where(sel0, idx, _BIG))
            selm = jnp.where(sel0 & (idx == cidx), f32(1.0), f32(0.0))
            bx1 = jnp.sum(x1_ref[...] * selm)
            by1 = jnp.sum(y1_ref[...] * selm)
            bx2 = jnp.sum(x2_ref[...] * selm)
            by2 = jnp.sum(y2_ref[...] * selm)
            bar = jnp.sum(ar_ref[...] * selm)

            row = jnp.where(ocol == 0, ms,
                  jnp.where(ocol == 1, bx1,
                  jnp.where(ocol == 2, by1,
                  jnp.where(ocol == 3, bx2,
                  jnp.where(ocol == 4, by2, f32(0.0))))))
            out_ref[...] = out_ref[...] + jnp.where(orow == i, row, f32(0.0))

            xx1 = jnp.maximum(bx1, x1_ref[...])
            yy1 = jnp.maximum(by1, y1_ref[...])
            xx2 = jnp.minimum(bx2, x2_ref[...])
            yy2 = jnp.minimum(by2, y2_ref[...])
            inter = jnp.maximum(xx2 - xx1, 0.0) * jnp.maximum(yy2 - yy1, 0.0)
            iou = inter / (bar + ar_ref[...] - inter + 1e-9)
            sw_ref[...] = jnp.where((iou > _THRESH) | (idx == cidx), neg_inf, sw)

        return 0

    jax.lax.fori_loop(0, _NMS_POST, _greedy, 0)


@jax.jit
def kernel(cls_scores, bbox_deltas):
    f32 = jnp.float32
    scores = cls_scores[0, 0::2, :, :].reshape(_N)
    scores = jnp.pad(scores, (0, _PAD - _N), constant_values=-jnp.inf)
    scores = scores.reshape(_ROWS, _COLS)

    e = (bbox_deltas[0].transpose(1, 2, 0)
         .reshape(_H, _W, _K, 4).reshape(_K, 4, _H, _W))
    dplanes = [
        jnp.pad(e[:, c, :, :].transpose(1, 2, 0).reshape(_N),
                (0, _PAD - _N)).reshape(_ROWS, _COLS)
        for c in range(4)
    ]
    aplanes = [jnp.asarray(p) for p in _ANCHOR_PLANES]

    out = pl.pallas_call(
        _nms_kernel,
        out_shape=jax.ShapeDtypeStruct((_NMS_POST, 8), f32),
        scratch_shapes=[pltpu.VMEM((_ROWS, _COLS), f32)] * 6,
    )(scores, *dplanes, *aplanes)
    return out[:, :5].reshape(1, _NMS_POST, 5)
